# Initial kernel scaffold; baseline (speedup 1.0000x reference)
import functools

import jax
import jax.numpy as jnp
from jax import lax
from jax.experimental import pallas as pl
from jax.experimental.pallas import tpu as pltpu
from jax.experimental.pallas import tpu_sc as plsc

N_ATOMS = 6_400_000
N_SEG = 100_000
NC, NS = 2, 16
NW = NC * NS
CPT = N_ATOMS // NW
CH = 4_000
NCHUNK = CPT // CH
ACC = 102_400
COLS = ACC // NW

_mesh = plsc.VectorSubcoreMesh(core_axis_name="c", subcore_axis_name="s")


@functools.partial(
    pl.kernel,
    out_type=jax.ShapeDtypeStruct((NW * ACC,), jnp.float32),
    mesh=_mesh,
    scratch_types=[
        pltpu.VMEM((ACC,), jnp.float32),
        pltpu.VMEM((CH,), jnp.float32),
        pltpu.VMEM((CH,), jnp.float32),
        pltpu.VMEM((CH,), jnp.int32),
        pltpu.VMEM((CH,), jnp.int32),
        pltpu.SemaphoreType.DMA,
        pltpu.SemaphoreType.DMA,
        pltpu.SemaphoreType.DMA,
        pltpu.SemaphoreType.DMA,
    ],
)
def _partial_sums(vals_hbm, idx_hbm, out_hbm, acc, v0, v1, i0, i1,
                  sv0, sv1, si0, si1):
    cid = lax.axis_index("c")
    sid = lax.axis_index("s")
    w = cid * NS + sid
    base = w * CPT

    zero16 = jnp.zeros((16,), jnp.float32)

    def _zero(k, carry):
        acc[pl.ds(k * 16, 16)] = zero16
        return carry

    lax.fori_loop(0, ACC // 16, _zero, 0)

    vbufs = (v0, v1)
    ibufs = (i0, i1)
    vsems = (sv0, sv1)
    isems = (si0, si1)

    handles = [None, None]
    handles[0] = (
        pltpu.async_copy(vals_hbm.at[pl.ds(base, CH)], v0, sv0),
        pltpu.async_copy(idx_hbm.at[pl.ds(base, CH)], i0, si0),
    )

    for c in range(NCHUNK):
        b = c & 1
        if c + 1 < NCHUNK:
            nb = b ^ 1
            off = base + (c + 1) * CH
            handles[nb] = (
                pltpu.async_copy(vals_hbm.at[pl.ds(off, CH)], vbufs[nb], vsems[nb]),
                pltpu.async_copy(idx_hbm.at[pl.ds(off, CH)], ibufs[nb], isems[nb]),
            )
        hv, hi = handles[b]
        hv.wait()
        hi.wait()
        vb = vbufs[b]
        ib = ibufs[b]

        def _scat(k, carry, vb=vb, ib=ib):
            idx = ib[pl.ds(k * 16, 16)]
            val = vb[pl.ds(k * 16, 16)]
            plsc.addupdate_scatter(acc, [idx], val)
            return carry

        lax.fori_loop(0, CH // 16, _scat, 0)

    pltpu.sync_copy(acc, out_hbm.at[pl.ds(w * ACC, ACC)])


@functools.partial(
    pl.kernel,
    out_type=jax.ShapeDtypeStruct((ACC,), jnp.float32),
    mesh=_mesh,
    scratch_types=[
        pltpu.VMEM((COLS,), jnp.float32),
        pltpu.VMEM((COLS,), jnp.float32),
        pltpu.SemaphoreType.DMA,
    ],
)
def _reduce_partials(p_hbm, out_hbm, sbuf, rbuf, sem):
    cid = lax.axis_index("c")
    sid = lax.axis_index("s")
    w = cid * NS + sid
    base = w * COLS

    pltpu.sync_copy(p_hbm.at[pl.ds(base, COLS)], sbuf)
    for r in range(1, NW):
        pltpu.sync_copy(p_hbm.at[pl.ds(r * ACC + base, COLS)], rbuf)

        def _add(k, carry):
            plsc.addupdate(sbuf.at[pl.ds(k * 16, 16)], rbuf[pl.ds(k * 16, 16)])
            return carry

        lax.fori_loop(0, COLS // 16, _add, 0)
    pltpu.sync_copy(sbuf, out_hbm.at[pl.ds(base, COLS)])


def kernel(atom_specific_values, index):
    idx32 = index.astype(jnp.int32)
    partials = _partial_sums(atom_specific_values, idx32)
    summed = _reduce_partials(partials)
    return summed[:N_SEG]


# speedup vs baseline: 20.6362x; 20.6362x over previous
import functools

import jax
import jax.numpy as jnp
from jax import lax
from jax.experimental import pallas as pl
from jax.experimental.pallas import tpu as pltpu
from jax.experimental.pallas import tpu_sc as plsc

N_ATOMS = 6_400_000
N_SEG = 100_000
NC, NS = 2, 16
NW = NC * NS
CPT = N_ATOMS // NW
CH = 4_000
NCHUNK = CPT // CH
ACC = 102_400
COLS = ACC // NW

_mesh = plsc.VectorSubcoreMesh(core_axis_name="c", subcore_axis_name="s")
_params = pltpu.CompilerParams(needs_layout_passes=False)


@functools.partial(
    pl.kernel,
    out_type=jax.ShapeDtypeStruct((NW * ACC,), jnp.float32),
    mesh=_mesh,
    compiler_params=_params,
    scratch_types=[
        pltpu.VMEM((ACC,), jnp.float32),
        pltpu.VMEM((CH,), jnp.float32),
        pltpu.VMEM((CH,), jnp.float32),
        pltpu.VMEM((CH,), jnp.int32),
        pltpu.VMEM((CH,), jnp.int32),
        pltpu.SemaphoreType.DMA,
        pltpu.SemaphoreType.DMA,
        pltpu.SemaphoreType.DMA,
        pltpu.SemaphoreType.DMA,
    ],
)
def _partial_sums(vals_hbm, idx_hbm, out_hbm, acc, v0, v1, i0, i1,
                  sv0, sv1, si0, si1):
    cid = lax.axis_index("c")
    sid = lax.axis_index("s")
    w = cid * NS + sid
    base = w * CPT

    zero16 = jnp.zeros((16,), jnp.float32)

    def _zero(k, carry):
        acc[pl.ds(k * 16, 16)] = zero16
        return carry

    lax.fori_loop(0, ACC // 16, _zero, 0)

    vbufs = (v0, v1)
    ibufs = (i0, i1)
    vsems = (sv0, sv1)
    isems = (si0, si1)

    handles = [None, None]
    handles[0] = (
        pltpu.async_copy(vals_hbm.at[pl.ds(base, CH)], v0, sv0),
        pltpu.async_copy(idx_hbm.at[pl.ds(base, CH)], i0, si0),
    )

    for c in range(NCHUNK):
        b = c & 1
        if c + 1 < NCHUNK:
            nb = b ^ 1
            off = base + (c + 1) * CH
            handles[nb] = (
                pltpu.async_copy(vals_hbm.at[pl.ds(off, CH)], vbufs[nb], vsems[nb]),
                pltpu.async_copy(idx_hbm.at[pl.ds(off, CH)], ibufs[nb], isems[nb]),
            )
        hv, hi = handles[b]
        hv.wait()
        hi.wait()
        vb = vbufs[b]
        ib = ibufs[b]

        def _scat(k, carry, vb=vb, ib=ib):
            idx = ib[pl.ds(k * 16, 16)]
            val = vb[pl.ds(k * 16, 16)]
            plsc.addupdate_scatter(acc, [idx], val)
            return carry

        lax.fori_loop(0, CH // 16, _scat, 0)

    pltpu.sync_copy(acc, out_hbm.at[pl.ds(w * ACC, ACC)])


@functools.partial(
    pl.kernel,
    out_type=jax.ShapeDtypeStruct((ACC,), jnp.float32),
    mesh=_mesh,
    compiler_params=_params,
    scratch_types=[
        pltpu.VMEM((COLS,), jnp.float32),
        pltpu.VMEM((COLS,), jnp.float32),
        pltpu.SemaphoreType.DMA,
    ],
)
def _reduce_partials(p_hbm, out_hbm, sbuf, rbuf, sem):
    cid = lax.axis_index("c")
    sid = lax.axis_index("s")
    w = cid * NS + sid
    base = w * COLS

    pltpu.sync_copy(p_hbm.at[pl.ds(base, COLS)], sbuf)
    for r in range(1, NW):
        pltpu.sync_copy(p_hbm.at[pl.ds(r * ACC + base, COLS)], rbuf)

        def _add(k, carry):
            plsc.addupdate(sbuf.at[pl.ds(k * 16, 16)], rbuf[pl.ds(k * 16, 16)])
            return carry

        lax.fori_loop(0, COLS // 16, _add, 0)
    pltpu.sync_copy(sbuf, out_hbm.at[pl.ds(base, COLS)])


def kernel(atom_specific_values, index):
    idx32 = index.astype(jnp.int32)
    partials = _partial_sums(atom_specific_values, idx32)
    summed = _reduce_partials(partials)
    return summed[:N_SEG]


# speedup vs baseline: 24.1688x; 1.1712x over previous
import functools

import jax
import jax.numpy as jnp
from jax import lax
from jax.experimental import pallas as pl
from jax.experimental.pallas import tpu as pltpu
from jax.experimental.pallas import tpu_sc as plsc

N_ATOMS = 6_400_000
N_SEG = 100_000
NC, NS = 2, 16
NW = NC * NS
CPT = N_ATOMS // NW
CH = 4_000
NCHUNK = CPT // CH
ACC = 102_400
COLS = ACC // NW

_mesh = plsc.VectorSubcoreMesh(core_axis_name="c", subcore_axis_name="s")
_params = pltpu.CompilerParams(needs_layout_passes=False)


@functools.partial(
    pl.kernel,
    out_type=jax.ShapeDtypeStruct((NW * ACC,), jnp.float32),
    mesh=_mesh,
    compiler_params=_params,
    scratch_types=[
        pltpu.VMEM((ACC,), jnp.float32),
        pltpu.VMEM((CH,), jnp.float32),
        pltpu.VMEM((CH,), jnp.float32),
        pltpu.VMEM((CH,), jnp.int32),
        pltpu.VMEM((CH,), jnp.int32),
        pltpu.SemaphoreType.DMA,
        pltpu.SemaphoreType.DMA,
        pltpu.SemaphoreType.DMA,
        pltpu.SemaphoreType.DMA,
    ],
)
def _partial_sums(vals_hbm, idx_hbm, out_hbm, acc, v0, v1, i0, i1,
                  sv0, sv1, si0, si1):
    cid = lax.axis_index("c")
    sid = lax.axis_index("s")
    w = cid * NS + sid
    base = w * CPT

    zero16 = jnp.zeros((16,), jnp.float32)

    def _zero(k, carry):
        for u in range(8):
            acc[pl.ds(k * 128 + u * 16, 16)] = zero16
        return carry

    lax.fori_loop(0, ACC // 128, _zero, 0)

    vbufs = (v0, v1)
    ibufs = (i0, i1)
    vsems = (sv0, sv1)
    isems = (si0, si1)

    handles = [None, None]
    handles[0] = (
        pltpu.async_copy(vals_hbm.at[pl.ds(base, CH)], v0, sv0),
        pltpu.async_copy(idx_hbm.at[pl.ds(base, CH)], i0, si0),
    )

    for c in range(NCHUNK):
        b = c & 1
        if c + 1 < NCHUNK:
            nb = b ^ 1
            off = base + (c + 1) * CH
            handles[nb] = (
                pltpu.async_copy(vals_hbm.at[pl.ds(off, CH)], vbufs[nb], vsems[nb]),
                pltpu.async_copy(idx_hbm.at[pl.ds(off, CH)], ibufs[nb], isems[nb]),
            )
        hv, hi = handles[b]
        hv.wait()
        hi.wait()
        vb = vbufs[b]
        ib = ibufs[b]

        def _scat(k, carry, vb=vb, ib=ib):
            for u in range(10):
                idx = ib[pl.ds(k * 160 + u * 16, 16)]
                val = vb[pl.ds(k * 160 + u * 16, 16)]
                plsc.addupdate_scatter(acc, [idx], val)
            return carry

        lax.fori_loop(0, CH // 160, _scat, 0)

    pltpu.sync_copy(acc, out_hbm.at[pl.ds(w * ACC, ACC)])


@functools.partial(
    pl.kernel,
    out_type=jax.ShapeDtypeStruct((ACC,), jnp.float32),
    mesh=_mesh,
    compiler_params=_params,
    scratch_types=[
        pltpu.VMEM((COLS,), jnp.float32),
        pltpu.VMEM((COLS,), jnp.float32),
        pltpu.VMEM((COLS,), jnp.float32),
        pltpu.SemaphoreType.DMA,
        pltpu.SemaphoreType.DMA,
    ],
)
def _reduce_partials(p_hbm, out_hbm, sbuf, r0, r1, s0, s1):
    cid = lax.axis_index("c")
    sid = lax.axis_index("s")
    w = cid * NS + sid
    base = w * COLS

    rbufs = (r0, r1)
    sems = (s0, s1)
    pltpu.sync_copy(p_hbm.at[pl.ds(base, COLS)], sbuf)
    handles = [None, None]
    handles[1] = pltpu.async_copy(p_hbm.at[pl.ds(ACC + base, COLS)], r1, s1)
    for r in range(1, NW):
        b = r & 1
        if r + 1 < NW:
            nb = b ^ 1
            handles[nb] = pltpu.async_copy(
                p_hbm.at[pl.ds((r + 1) * ACC + base, COLS)], rbufs[nb], sems[nb])
        handles[b].wait()
        rbuf = rbufs[b]

        def _add(k, carry, rbuf=rbuf):
            for u in range(8):
                plsc.addupdate(sbuf.at[pl.ds(k * 128 + u * 16, 16)],
                               rbuf[pl.ds(k * 128 + u * 16, 16)])
            return carry

        lax.fori_loop(0, COLS // 128, _add, 0)
    pltpu.sync_copy(sbuf, out_hbm.at[pl.ds(base, COLS)])


def kernel(atom_specific_values, index):
    idx32 = index.astype(jnp.int32)
    partials = _partial_sums(atom_specific_values, idx32)
    summed = _reduce_partials(partials)
    return summed[:N_SEG]


# speedup vs baseline: 54.8530x; 2.2696x over previous
import functools

import jax
import jax.numpy as jnp
from jax import lax
from jax.experimental import pallas as pl
from jax.experimental.pallas import tpu as pltpu
from jax.experimental.pallas import tpu_sc as plsc

N_ATOMS = 6_400_000
N_SEG = 100_000
NC, NS = 2, 16
NW = NC * NS
CPT = N_ATOMS // NW
CH = 4_000
NCHUNK = CPT // CH
ACC = 102_400
COLS = ACC // NW

_mesh = plsc.VectorSubcoreMesh(core_axis_name="c", subcore_axis_name="s")
_params = pltpu.CompilerParams(needs_layout_passes=False)


@functools.partial(
    pl.kernel,
    out_type=jax.ShapeDtypeStruct((NW * ACC,), jnp.float32),
    mesh=_mesh,
    compiler_params=_params,
    scratch_types=[
        pltpu.VMEM((ACC,), jnp.float32),
        pltpu.VMEM((CH,), jnp.float32),
        pltpu.VMEM((CH,), jnp.float32),
        pltpu.VMEM((CH,), jnp.int32),
        pltpu.VMEM((CH,), jnp.int32),
        pltpu.SemaphoreType.DMA,
        pltpu.SemaphoreType.DMA,
        pltpu.SemaphoreType.DMA,
        pltpu.SemaphoreType.DMA,
    ],
)
def _partial_sums(vals_hbm, idx_hbm, out_hbm, acc, v0, v1, i0, i1,
                  sv0, sv1, si0, si1):
    cid = lax.axis_index("c")
    sid = lax.axis_index("s")
    w = cid * NS + sid
    base = w * CPT

    zero16 = jnp.zeros((16,), jnp.float32)

    def _zero(k, carry):
        for u in range(8):
            acc[pl.ds(k * 128 + u * 16, 16)] = zero16
        return carry

    lax.fori_loop(0, ACC // 128, _zero, 0)

    vbufs = (v0, v1)
    ibufs = (i0, i1)
    vsems = (sv0, sv1)
    isems = (si0, si1)

    handles = [None, None]
    handles[0] = (
        pltpu.async_copy(vals_hbm.at[pl.ds(base, CH)], v0, sv0),
        pltpu.async_copy(idx_hbm.at[pl.ds(base, CH)], i0, si0),
    )

    for c in range(NCHUNK):
        b = c & 1
        if c + 1 < NCHUNK:
            nb = b ^ 1
            off = base + (c + 1) * CH
            handles[nb] = (
                pltpu.async_copy(vals_hbm.at[pl.ds(off, CH)], vbufs[nb], vsems[nb]),
                pltpu.async_copy(idx_hbm.at[pl.ds(off, CH)], ibufs[nb], isems[nb]),
            )
        hv, hi = handles[b]
        hv.wait()
        hi.wait()
        vb = vbufs[b]
        ib = ibufs[b]

        lane_base = lax.iota(jnp.int32, 16) * (CH // 16)

        def _scat(k, carry, vb=vb, ib=ib):
            for u in range(10):
                lidx = lane_base + (k * 10 + u)
                idx = plsc.load_gather(ib, [lidx])
                val = plsc.load_gather(vb, [lidx])
                plsc.addupdate_scatter(acc, [idx], val)
            return carry

        lax.fori_loop(0, CH // 160, _scat, 0)

    pltpu.sync_copy(acc, out_hbm.at[pl.ds(w * ACC, ACC)])


@functools.partial(
    pl.kernel,
    out_type=jax.ShapeDtypeStruct((ACC,), jnp.float32),
    mesh=_mesh,
    compiler_params=_params,
    scratch_types=[
        pltpu.VMEM((COLS,), jnp.float32),
        pltpu.VMEM((COLS,), jnp.float32),
        pltpu.VMEM((COLS,), jnp.float32),
        pltpu.SemaphoreType.DMA,
        pltpu.SemaphoreType.DMA,
    ],
)
def _reduce_partials(p_hbm, out_hbm, sbuf, r0, r1, s0, s1):
    cid = lax.axis_index("c")
    sid = lax.axis_index("s")
    w = cid * NS + sid
    base = w * COLS

    rbufs = (r0, r1)
    sems = (s0, s1)
    pltpu.sync_copy(p_hbm.at[pl.ds(base, COLS)], sbuf)
    handles = [None, None]
    handles[1] = pltpu.async_copy(p_hbm.at[pl.ds(ACC + base, COLS)], r1, s1)
    for r in range(1, NW):
        b = r & 1
        if r + 1 < NW:
            nb = b ^ 1
            handles[nb] = pltpu.async_copy(
                p_hbm.at[pl.ds((r + 1) * ACC + base, COLS)], rbufs[nb], sems[nb])
        handles[b].wait()
        rbuf = rbufs[b]

        def _add(k, carry, rbuf=rbuf):
            for u in range(8):
                plsc.addupdate(sbuf.at[pl.ds(k * 128 + u * 16, 16)],
                               rbuf[pl.ds(k * 128 + u * 16, 16)])
            return carry

        lax.fori_loop(0, COLS // 128, _add, 0)
    pltpu.sync_copy(sbuf, out_hbm.at[pl.ds(base, COLS)])


def kernel(atom_specific_values, index):
    idx32 = index.astype(jnp.int32)
    partials = _partial_sums(atom_specific_values, idx32)
    summed = _reduce_partials(partials)
    return summed[:N_SEG]
